# Initial kernel scaffold; baseline (speedup 1.0000x reference)
import jax
import jax.numpy as jnp
from jax.experimental import pallas as pl


def kernel(query_up, key_up, W0, W1, head_idx):
    raise NotImplementedError("write your pallas kernel here")


# speedup vs baseline: 11.8860x; 11.8860x over previous
import functools

import jax
import jax.numpy as jnp
from jax import lax
from jax.experimental import pallas as pl

_B, _S, _D = 2, 2048, 64
_K_MAX = 64
_QB = 256


def _signatures(x, w):
    proj = lax.dot_general(x, w, (((1,), (0,)), ((), ())))
    h = jnp.mod(jnp.floor(proj / 4.0), 32.0).astype(jnp.int32)
    bits = (x > 0).astype(jnp.int32)
    shifts = lax.broadcasted_iota(jnp.int32, (1, 32), 1)
    code0 = jnp.sum(bits[:, :32] << shifts, axis=1)
    code1 = jnp.sum(bits[:, 32:] << shifts, axis=1)
    return code0, code1, h


def _finder_block(q_ref, k_ref, w_ref, out_ref):
    q = q_ref[0]
    k = k_ref[0]
    w = w_ref[...]

    qc0, qc1, qh = _signatures(q, w)
    kc0, kc1, kh = _signatures(k, w)

    def group_mask(qc, kc, hlo):
        code_eq = qc[:, None] == kc[None, :]
        hm = (qh[:, hlo, None] == kh[None, :, hlo])
        for j in range(hlo + 1, hlo + 4):
            hm = hm | (qh[:, j, None] == kh[None, :, j])
        return code_eq & hm

    mask = group_mask(qc0, kc0, 0) | group_mask(qc1, kc1, 4)

    scores = lax.dot_general(q, k, (((1,), (1,)), ((), ())))
    masked = jnp.where(mask, scores, jnp.float32(-1e9))

    n_cand = jnp.max(jnp.sum(mask.astype(jnp.int32), axis=1))
    trip = jnp.minimum(n_cand, _K_MAX)

    kidx = lax.broadcasted_iota(jnp.int32, (_QB, _S), 1)
    slot = lax.broadcasted_iota(jnp.int32, (_QB, _K_MAX), 1)
    out0 = jnp.full((_QB, _K_MAX), -1, jnp.int32)

    def body(i, carry):
        m, out = carry
        row_max = jnp.max(m, axis=1)
        at_max = m == row_max[:, None]
        pick = jnp.min(jnp.where(at_max, kidx, jnp.int32(_S)), axis=1)
        valid = row_max > jnp.float32(-1e8)
        col = jnp.where(valid, pick, -1)
        out = jnp.where(slot == i, col[:, None], out)
        m = jnp.where((kidx == pick[:, None]) & valid[:, None],
                      jnp.float32(-1e9), m)
        return m, out

    _, out = lax.fori_loop(0, trip, body, (masked, out0))
    out_ref[0] = out


def _finder(query_up, key_up, wc):
    grid = (_B, _S // _QB)
    return pl.pallas_call(
        _finder_block,
        grid=grid,
        in_specs=[
            pl.BlockSpec((1, _QB, _D), lambda b, i: (b, i, 0)),
            pl.BlockSpec((1, _S, _D), lambda b, i: (b, 0, 0)),
            pl.BlockSpec((_D, 8), lambda b, i: (0, 0)),
        ],
        out_specs=pl.BlockSpec((1, _QB, _K_MAX), lambda b, i: (b, i, 0)),
        out_shape=jax.ShapeDtypeStruct((_B, _S, _K_MAX), jnp.int32),
    )(query_up, key_up, wc)


def kernel(query_up, key_up, W0, W1, head_idx=0):
    wc = jnp.zeros((_D, 8), jnp.float32)
    wc = wc.at[:32, :4].set(W0).at[32:, 4:].set(W1)
    return _finder(query_up, key_up, wc)
